# baseline (speedup 1.0000x reference)
import functools

import jax
import jax.numpy as jnp
from jax import lax
from jax.experimental import pallas as pl
from jax.experimental.pallas import tpu as pltpu
from jax.experimental.pallas import tpu_sc as plsc

N = 10000
D = 128
E = 320000
NC, NS, L = 2, 16, 16
NW = NC * NS
EPW = E // NW
C = 128
NFULL = EPW // C
TAIL = EPW - NFULL * C
NPAD = 10240
RPT = NPAD // NS
K = 80
KH = K // 2
PADE = NW * K * C
DUMMY = N + 16

_MESH = plsc.VectorSubcoreMesh(core_axis_name="c", subcore_axis_name="s")


def _worker_id():
    return lax.axis_index("s") * NC + lax.axis_index("c")


def _deg_body(dst_hbm, deg_part, ones_v, dix_v, gath_v, acc_sh):
    cid = lax.axis_index("c")
    sid = lax.axis_index("s")
    wid = _worker_id()

    one16 = jnp.full((L,), 1.0, jnp.float32)
    zero16 = jnp.zeros((L,), jnp.float32)

    def fill(r, _):
        ones_v[r, :] = one16
        gath_v[r, :] = zero16
        return 0
    lax.fori_loop(0, C, fill, 0)

    for z in range(RPT // C):
        pltpu.sync_copy(gath_v, acc_sh.at[pl.ds(sid * RPT + z * C, C), :])
    plsc.subcore_barrier()

    ebase = wid * K * C

    def chunk(c, _):
        pltpu.sync_copy(dst_hbm.at[pl.ds(ebase + c * C, C)], dix_v)
        pltpu.sync_copy(ones_v, acc_sh.at[dix_v], add=True)
        return 0
    lax.fori_loop(0, K, chunk, 0)
    plsc.subcore_barrier()

    for z in range(RPT // C):
        sl = pl.ds(sid * RPT + z * C, C)
        pltpu.sync_copy(acc_sh.at[sl, :], gath_v)
        pltpu.sync_copy(gath_v, deg_part.at[cid, sl, :])


_deg_call = pl.kernel(
    _deg_body,
    out_type=jax.ShapeDtypeStruct((NC, NPAD, L), jnp.float32),
    mesh=_MESH,
    scratch_types=[
        pltpu.VMEM((C, L), jnp.float32),
        pltpu.VMEM((C,), jnp.int32),
        pltpu.VMEM((C, L), jnp.float32),
        pltpu.VMEM_SHARED((NPAD, L), jnp.float32),
    ],
)


def _agg_body(g_hbm, src_hbm, dst_hbm, part, six_a, six_b, dix_v, rows_a,
              rows_b, acc_sh, sem_a, sem_b):
    cid = lax.axis_index("c")
    sid = lax.axis_index("s")
    wid = _worker_id()

    zero16 = jnp.zeros((L,), jnp.float32)

    def fill(r, _):
        for k in range(D // L):
            rows_a[r, pl.ds(k * L, L)] = zero16
        return 0
    lax.fori_loop(0, C, fill, 0)

    for z in range(RPT // C):
        pltpu.sync_copy(rows_a, acc_sh.at[pl.ds(sid * RPT + z * C, C), :])
    plsc.subcore_barrier()

    ebase = wid * K * C

    def sload(c, buf):
        pltpu.sync_copy(src_hbm.at[pl.ds(ebase + c * C, C)], buf)

    def gstart(buf, rows, sem):
        pltpu.async_copy(g_hbm.at[buf], rows, sem)

    def gwait(buf, rows, sem):
        pltpu.make_async_copy(g_hbm.at[buf], rows, sem).wait()

    def dscat(c, rows):
        pltpu.sync_copy(dst_hbm.at[pl.ds(ebase + c * C, C)], dix_v)
        pltpu.sync_copy(rows, acc_sh.at[dix_v], add=True)

    sload(0, six_a)
    gstart(six_a, rows_a, sem_a)

    def body(i, _):
        ca = 2 * i
        cb = ca + 1
        sload(cb, six_b)
        gstart(six_b, rows_b, sem_b)
        gwait(six_a, rows_a, sem_a)
        dscat(ca, rows_a)

        @pl.when(i < KH - 1)
        def _():
            sload(ca + 2, six_a)
            gstart(six_a, rows_a, sem_a)

        gwait(six_b, rows_b, sem_b)
        dscat(cb, rows_b)
        return 0
    lax.fori_loop(0, KH, body, 0)
    plsc.subcore_barrier()

    for z in range(RPT // C):
        sl = pl.ds(sid * RPT + z * C, C)
        rows = rows_a if z % 2 == 0 else rows_b
        sem = sem_a if z % 2 == 0 else sem_b
        if z >= 2:
            psl = pl.ds(sid * RPT + (z - 2) * C, C)
            pltpu.make_async_copy(rows, part.at[cid, psl, :], sem).wait()
        pltpu.sync_copy(acc_sh.at[sl, :], rows)
        pltpu.async_copy(rows, part.at[cid, sl, :], sem)
    for z in (RPT // C - 2, RPT // C - 1):
        sl = pl.ds(sid * RPT + z * C, C)
        rows = rows_a if z % 2 == 0 else rows_b
        sem = sem_a if z % 2 == 0 else sem_b
        pltpu.make_async_copy(rows, part.at[cid, sl, :], sem).wait()


_agg_call = pl.kernel(
    _agg_body,
    out_type=jax.ShapeDtypeStruct((NC, NPAD, D), jnp.float32),
    mesh=_MESH,
    scratch_types=[
        pltpu.VMEM((C,), jnp.int32),
        pltpu.VMEM((C,), jnp.int32),
        pltpu.VMEM((C,), jnp.int32),
        pltpu.VMEM((C, D), jnp.float32),
        pltpu.VMEM((C, D), jnp.float32),
        pltpu.VMEM_SHARED((NPAD, D), jnp.float32),
        pltpu.SemaphoreType.DMA,
        pltpu.SemaphoreType.DMA,
    ],
)


def _dinv_body(degp_ref, o_ref):
    deg = degp_ref[0, :, 0:1] + degp_ref[1, :, 0:1] + 1.0
    o_ref[...] = lax.rsqrt(deg)


_dinv_call = pl.pallas_call(
    _dinv_body,
    out_shape=jax.ShapeDtypeStruct((NPAD, 1), jnp.float32),
)

_RB = 2000
_GRID = N // _RB


def _mm1_body(d_ref, x_ref, w_ref, o_ref):
    h = jnp.dot(x_ref[...], w_ref[...], preferred_element_type=jnp.float32)
    o_ref[...] = d_ref[...] * h


_mm1_call = pl.pallas_call(
    _mm1_body,
    grid=(_GRID,),
    in_specs=[
        pl.BlockSpec((_RB, 1), lambda i: (i, 0)),
        pl.BlockSpec((_RB, D), lambda i: (i, 0)),
        pl.BlockSpec((D, D), lambda i: (0, 0)),
    ],
    out_specs=pl.BlockSpec((_RB, D), lambda i: (i, 0)),
    out_shape=jax.ShapeDtypeStruct((N, D), jnp.float32),
)


def _mid_body(p_ref, g_ref, d_ref, b_ref, w_ref, o_ref):
    agg = p_ref[0] + p_ref[1]
    z = jnp.maximum(d_ref[...] * (agg + g_ref[...]) + b_ref[...], 0.0)
    o_ref[...] = d_ref[...] * jnp.dot(
        z, w_ref[...], preferred_element_type=jnp.float32)


_mid_call = pl.pallas_call(
    _mid_body,
    grid=(_GRID,),
    in_specs=[
        pl.BlockSpec((NC, _RB, D), lambda i: (0, i, 0)),
        pl.BlockSpec((_RB, D), lambda i: (i, 0)),
        pl.BlockSpec((_RB, 1), lambda i: (i, 0)),
        pl.BlockSpec((1, D), lambda i: (0, 0)),
        pl.BlockSpec((D, D), lambda i: (0, 0)),
    ],
    out_specs=pl.BlockSpec((_RB, D), lambda i: (i, 0)),
    out_shape=jax.ShapeDtypeStruct((N, D), jnp.float32),
)


def _fin_body(q_ref, g_ref, d_ref, b_ref, o_ref):
    agg = q_ref[0] + q_ref[1]
    o_ref[...] = d_ref[...] * (agg + g_ref[...]) + b_ref[...]


_fin_call = pl.pallas_call(
    _fin_body,
    grid=(_GRID,),
    in_specs=[
        pl.BlockSpec((NC, _RB, D), lambda i: (0, i, 0)),
        pl.BlockSpec((_RB, D), lambda i: (i, 0)),
        pl.BlockSpec((_RB, 1), lambda i: (i, 0)),
        pl.BlockSpec((1, D), lambda i: (0, 0)),
    ],
    out_specs=pl.BlockSpec((_RB, D), lambda i: (i, 0)),
    out_shape=jax.ShapeDtypeStruct((N, D), jnp.float32),
)


@jax.jit
def kernel(x, edge_index, W1, b1, W2, b2):
    src = edge_index[0].astype(jnp.int32)
    dst = edge_index[1].astype(jnp.int32)
    src1 = jnp.concatenate([src, jnp.zeros((PADE - E,), jnp.int32)])
    dst1 = jnp.concatenate(
        [dst, jnp.full((PADE - E,), DUMMY, jnp.int32)])

    deg_part = _deg_call(dst1)
    dcol = _dinv_call(deg_part)[:N]

    b1r = b1.reshape(1, D)
    b2r = b2.reshape(1, D)

    g1 = _mm1_call(dcol, x, W1)
    p = _agg_call(g1, src1, dst1)
    g2 = _mid_call(p[:, :N], g1, dcol, b1r, W2)
    q = _agg_call(g2, src1, dst1)
    return _fin_call(q[:, :N], g2, dcol, b2r)


# speedup vs baseline: 1.0018x; 1.0018x over previous
import functools

import jax
import jax.numpy as jnp
from jax import lax
from jax.experimental import pallas as pl
from jax.experimental.pallas import tpu as pltpu
from jax.experimental.pallas import tpu_sc as plsc

N = 10000
D = 128
E = 320000
NC, NS, L = 2, 16, 16
NW = NC * NS
EPW = E // NW
C = 128
NFULL = EPW // C
TAIL = EPW - NFULL * C
NPAD = 10240
RPT = NPAD // NS
K = 80
KH = K // 2
PADE = NW * K * C
DUMMY = N + 16

_MESH = plsc.VectorSubcoreMesh(core_axis_name="c", subcore_axis_name="s")


def _worker_id():
    return lax.axis_index("s") * NC + lax.axis_index("c")


def _deg_body(dst_hbm, deg_part, ones_v, dix_v, gath_v, acc_sh):
    cid = lax.axis_index("c")
    sid = lax.axis_index("s")
    wid = _worker_id()

    one16 = jnp.full((L,), 1.0, jnp.float32)
    zero16 = jnp.zeros((L,), jnp.float32)

    def fill(r, _):
        ones_v[r, :] = one16
        gath_v[r, :] = zero16
        return 0
    lax.fori_loop(0, C, fill, 0)

    for z in range(RPT // C):
        pltpu.sync_copy(gath_v, acc_sh.at[pl.ds(sid * RPT + z * C, C), :])
    plsc.subcore_barrier()

    ebase = wid * K * C

    def chunk(c, _):
        pltpu.sync_copy(dst_hbm.at[pl.ds(ebase + c * C, C)], dix_v)
        pltpu.sync_copy(ones_v, acc_sh.at[dix_v], add=True)
        return 0
    lax.fori_loop(0, K, chunk, 0)
    plsc.subcore_barrier()

    for z in range(RPT // C):
        sl = pl.ds(sid * RPT + z * C, C)
        pltpu.sync_copy(acc_sh.at[sl, :], gath_v)
        pltpu.sync_copy(gath_v, deg_part.at[cid, sl, :])


_deg_call = pl.kernel(
    _deg_body,
    out_type=jax.ShapeDtypeStruct((NC, NPAD, L), jnp.float32),
    mesh=_MESH,
    scratch_types=[
        pltpu.VMEM((C, L), jnp.float32),
        pltpu.VMEM((C,), jnp.int32),
        pltpu.VMEM((C, L), jnp.float32),
        pltpu.VMEM_SHARED((NPAD, L), jnp.float32),
    ],
)


def _agg_body(g_hbm, src_hbm, dst_hbm, part, six_a, six_b, dix_v, rows_a,
              rows_b, acc_sh, sem_a, sem_b):
    cid = lax.axis_index("c")
    sid = lax.axis_index("s")
    wid = _worker_id()

    zero16 = jnp.zeros((L,), jnp.float32)

    def fill(r, _):
        for k in range(D // L):
            rows_a[r, pl.ds(k * L, L)] = zero16
        return 0
    lax.fori_loop(0, C, fill, 0)

    for z in range(RPT // C):
        pltpu.sync_copy(rows_a, acc_sh.at[pl.ds(sid * RPT + z * C, C), :])
    plsc.subcore_barrier()

    ebase = wid * K * C

    def sload(c, buf):
        pltpu.sync_copy(src_hbm.at[pl.ds(ebase + c * C, C)], buf)

    def gstart(buf, rows, sem):
        pltpu.async_copy(g_hbm.at[buf], rows, sem)

    def gwait(buf, rows, sem):
        pltpu.make_async_copy(g_hbm.at[buf], rows, sem).wait()

    def dscat(c, rows):
        pltpu.sync_copy(dst_hbm.at[pl.ds(ebase + c * C, C)], dix_v)
        pltpu.sync_copy(rows, acc_sh.at[dix_v], add=True)

    sload(0, six_a)
    gstart(six_a, rows_a, sem_a)

    def body(i, _):
        ca = 2 * i
        cb = ca + 1
        sload(cb, six_b)
        gstart(six_b, rows_b, sem_b)
        gwait(six_a, rows_a, sem_a)
        dscat(ca, rows_a)

        @pl.when(i < KH - 1)
        def _():
            sload(ca + 2, six_a)
            gstart(six_a, rows_a, sem_a)

        gwait(six_b, rows_b, sem_b)
        dscat(cb, rows_b)
        return 0
    lax.fori_loop(0, KH, body, 0)
    plsc.subcore_barrier()

    for z in range(RPT // C):
        sl = pl.ds(sid * RPT + z * C, C)
        rows = rows_a if z % 2 == 0 else rows_b
        sem = sem_a if z % 2 == 0 else sem_b
        if z >= 2:
            psl = pl.ds(sid * RPT + (z - 2) * C, C)
            pltpu.make_async_copy(rows, part.at[cid, psl, :], sem).wait()
        pltpu.sync_copy(acc_sh.at[sl, :], rows)
        pltpu.async_copy(rows, part.at[cid, sl, :], sem)
    for z in (RPT // C - 2, RPT // C - 1):
        sl = pl.ds(sid * RPT + z * C, C)
        rows = rows_a if z % 2 == 0 else rows_b
        sem = sem_a if z % 2 == 0 else sem_b
        pltpu.make_async_copy(rows, part.at[cid, sl, :], sem).wait()


_agg_call = pl.kernel(
    _agg_body,
    out_type=jax.ShapeDtypeStruct((NC, NPAD, D), jnp.float32),
    mesh=_MESH,
    scratch_types=[
        pltpu.VMEM((C,), jnp.int32),
        pltpu.VMEM((C,), jnp.int32),
        pltpu.VMEM((C,), jnp.int32),
        pltpu.VMEM((C, D), jnp.float32),
        pltpu.VMEM((C, D), jnp.float32),
        pltpu.VMEM_SHARED((NPAD, D), jnp.float32),
        pltpu.SemaphoreType.DMA,
        pltpu.SemaphoreType.DMA,
    ],
)


def _dinv_body(degp_ref, o_ref):
    deg = degp_ref[0, :, 0:1] + degp_ref[1, :, 0:1] + 1.0
    o_ref[...] = lax.rsqrt(deg)


_dinv_call = pl.pallas_call(
    _dinv_body,
    out_shape=jax.ShapeDtypeStruct((NPAD, 1), jnp.float32),
)

_RB = 2000
_GRID = N // _RB


def _mm1_body(d_ref, x_ref, w_ref, o_ref):
    h = jnp.dot(x_ref[...], w_ref[...], preferred_element_type=jnp.float32)
    o_ref[...] = d_ref[...] * h


_mm1_call = pl.pallas_call(
    _mm1_body,
    grid=(_GRID,),
    in_specs=[
        pl.BlockSpec((_RB, 1), lambda i: (i, 0)),
        pl.BlockSpec((_RB, D), lambda i: (i, 0)),
        pl.BlockSpec((D, D), lambda i: (0, 0)),
    ],
    out_specs=pl.BlockSpec((_RB, D), lambda i: (i, 0)),
    out_shape=jax.ShapeDtypeStruct((N, D), jnp.float32),
)


def _mid_body(p_ref, g_ref, d_ref, b_ref, w_ref, o_ref):
    agg = p_ref[0] + p_ref[1]
    z = jnp.maximum(d_ref[...] * (agg + g_ref[...]) + b_ref[...], 0.0)
    o_ref[...] = d_ref[...] * jnp.dot(
        z, w_ref[...], preferred_element_type=jnp.float32)


_mid_call = pl.pallas_call(
    _mid_body,
    grid=(_GRID,),
    in_specs=[
        pl.BlockSpec((NC, _RB, D), lambda i: (0, i, 0)),
        pl.BlockSpec((_RB, D), lambda i: (i, 0)),
        pl.BlockSpec((_RB, 1), lambda i: (i, 0)),
        pl.BlockSpec((1, D), lambda i: (0, 0)),
        pl.BlockSpec((D, D), lambda i: (0, 0)),
    ],
    out_specs=pl.BlockSpec((_RB, D), lambda i: (i, 0)),
    out_shape=jax.ShapeDtypeStruct((N, D), jnp.float32),
)


def _fin_body(q_ref, g_ref, d_ref, b_ref, o_ref):
    agg = q_ref[0] + q_ref[1]
    o_ref[...] = d_ref[...] * (agg + g_ref[...]) + b_ref[...]


_fin_call = pl.pallas_call(
    _fin_body,
    grid=(_GRID,),
    in_specs=[
        pl.BlockSpec((NC, _RB, D), lambda i: (0, i, 0)),
        pl.BlockSpec((_RB, D), lambda i: (i, 0)),
        pl.BlockSpec((_RB, 1), lambda i: (i, 0)),
        pl.BlockSpec((1, D), lambda i: (0, 0)),
    ],
    out_specs=pl.BlockSpec((_RB, D), lambda i: (i, 0)),
    out_shape=jax.ShapeDtypeStruct((N, D), jnp.float32),
)


@jax.jit
def kernel(x, edge_index, W1, b1, W2, b2):
    src = edge_index[0].astype(jnp.int32)
    dst = edge_index[1].astype(jnp.int32)
    src1 = jnp.concatenate([src, jnp.zeros((PADE - E,), jnp.int32)])
    dummy = N + jnp.arange(PADE - E, dtype=jnp.int32) % (NPAD - N)
    dst1 = jnp.concatenate([dst, dummy])

    deg_part = _deg_call(dst1)
    dcol = _dinv_call(deg_part)[:N]

    b1r = b1.reshape(1, D)
    b2r = b2.reshape(1, D)

    g1 = _mm1_call(dcol, x, W1)
    p = _agg_call(g1, src1, dst1)
    g2 = _mid_call(p[:, :N], g1, dcol, b1r, W2)
    q = _agg_call(g2, src1, dst1)
    return _fin_call(q[:, :N], g2, dcol, b2r)
